# baseline (speedup 1.0000x reference)
import jax
import jax.numpy as jnp
from jax import lax
from jax.experimental import pallas as pl
from jax.experimental.pallas import tpu as pltpu
from jax.experimental.pallas import tpu_sc as plsc

B, L = 4096, 50
DTOT = 112
TJ = DTOT // 8

_info = plsc.get_sparse_core_info()
NC, NS = _info.num_cores, _info.num_subcores
NW = NC * NS


def _sc_body(mid_t, uid_t, cid_t, mcc_t, cty_t,
             mid_r, uid_r, cid_r, mcc_r, cty_r,
             Wm, Wu, Wct, Wmcct, Wctyt,
             out, iv0, iv1, iv2, iv3, iv4, tc_v, tmcc_v, tcty_v,
             bmA, buA, bmB, buB, tileA, tileB, gsemA, gsemB, wsemA, wsemB):
    wid = lax.axis_index("s") * NC + lax.axis_index("c")
    bcol = pl.ds(wid * 128, 128)

    for nat, rem, iv in ((mid_t, mid_r, iv0), (uid_t, uid_r, iv1),
                         (cid_t, cid_r, iv2), (mcc_t, mcc_r, iv3),
                         (cty_t, cty_r, iv4)):
        pltpu.sync_copy(nat.at[:, wid], iv.at[pl.ds(0, 6)])
        pltpu.sync_copy(rem.at[0, bcol], iv.at[6, 0])
        pltpu.sync_copy(rem.at[1, bcol], iv.at[6, 1])

    for iv in (iv0, iv1):
        def scale(p, carry, iv=iv):
            for r in range(8):
                for k in range(8):
                    s = pl.ds(k * 16, 16)
                    iv[p, r, s] = iv[p, r, s] * 4
            return carry
        lax.fori_loop(0, 7, scale, 0)
    pltpu.sync_copy(Wct, tc_v)
    pltpu.sync_copy(Wmcct, tmcc_v)
    pltpu.sync_copy(Wctyt, tcty_v)

    lane = lax.iota(jnp.int32, 16)
    smalls = (tc_v, tmcc_v, tcty_v)
    siv = (iv2, iv3, iv4)

    def issue_gathers(l, bm, bu, gsem):
        pltpu.async_copy(Wm.at[iv0.at[l >> 3, l & 7]], bm, gsem)
        pltpu.async_copy(Wu.at[iv1.at[l >> 3, l & 7]], bu, gsem)

    def wait_gathers(l, bm, bu, gsem):
        pltpu.make_async_copy(Wm.at[iv0.at[l >> 3, l & 7]], bm, gsem).wait()
        pltpu.make_async_copy(Wu.at[iv1.at[l >> 3, l & 7]], bu, gsem).wait()

    rots = [(lane + j) & 15 for j in range(16)]

    def transpose_into(tile, bm, bu, l):
        def bchunk(bc, carry):
            row = bc * 16 + lane
            for tjoff, buf in ((0, bm), (4, bu)):
                for f0 in (0, 16):
                    for j in range(16):
                        f = f0 + rots[j]
                        v = plsc.load_gather(buf, [row, f])
                        plsc.store_scatter(
                            tile, [tjoff + (f >> 3), f & 7, row], v)
            return carry
        lax.fori_loop(0, 8, bchunk, 0)

        def chunk(k, carry):
            col16 = pl.ds(k * 16, 16)
            for t in range(3):
                idxv = siv[t][l >> 3, l & 7, pl.ds(k * 16, 16)]
                for f in range(16):
                    v = plsc.load_gather(
                        smalls[t], [jnp.full((16,), f, jnp.int32), idxv])
                    tile[8 + 2 * t + f // 8, f % 8, col16] = v
            return carry
        lax.fori_loop(0, 8, chunk, 0)

    def issue_write(l, tile, wsem):
        pltpu.async_copy(tile, out.at[l, :, wid], wsem)

    def wait_write(l, tile, wsem):
        pltpu.make_async_copy(tile, out.at[l, :, wid], wsem).wait()

    issue_gathers(0, bmA, buA, gsemA)
    issue_gathers(1, bmB, buB, gsemB)

    def step(k, carry):
        l0, l1 = 2 * k, 2 * k + 1

        wait_gathers(l0, bmA, buA, gsemA)

        @pl.when(k > 0)
        def _():
            wait_write(l0, tileA, wsemA)
        transpose_into(tileA, bmA, buA, l0)
        issue_write(l0, tileA, wsemA)

        @pl.when(k < L // 2 - 1)
        def _():
            issue_gathers(l0 + 2, bmA, buA, gsemA)

        wait_gathers(l1, bmB, buB, gsemB)

        @pl.when(k > 0)
        def _():
            wait_write(l1, tileB, wsemB)
        transpose_into(tileB, bmB, buB, l1)
        issue_write(l1, tileB, wsemB)

        @pl.when(k < L // 2 - 1)
        def _():
            issue_gathers(l1 + 2, bmB, buB, gsemB)
        return carry

    lax.fori_loop(0, L // 2, step, 0)
    wait_write(L - 2, tileA, wsemA)
    wait_write(L - 1, tileB, wsemB)


@jax.jit
def kernel(merchant_id, user_id, category_id, mcc, country,
           W_merchant_id, W_user_id, W_category_id, W_mcc, W_country):
    mesh = plsc.VectorSubcoreMesh(core_axis_name="c", subcore_axis_name="s")
    run = pl.kernel(
        _sc_body,
        out_type=jax.ShapeDtypeStruct((L, TJ, NW, 8, 128), jnp.float32),
        mesh=mesh,
        scratch_types=(
            [pltpu.VMEM((7, 8, 128), jnp.int32) for _ in range(5)]
            + [pltpu.VMEM((16, 1000), jnp.float32) for _ in range(3)]
            + [pltpu.VMEM((128, 32), jnp.float32) for _ in range(4)]
            + [pltpu.VMEM((TJ, 8, 128), jnp.float32) for _ in range(2)]
            + [pltpu.SemaphoreType.DMA] * 4
        ),
        compiler_params=pltpu.CompilerParams(use_tc_tiling_on_sc=False,
                                             needs_layout_passes=False),
    )
    def native_view(x):
        return x[:, :48].reshape(B // 128, 128, 6, 8).transpose(2, 0, 3, 1)

    def rem_view(x):
        return x[:, 48:50].T

    out5d = run(
        native_view(merchant_id), native_view(user_id),
        native_view(category_id), native_view(mcc), native_view(country),
        rem_view(merchant_id), rem_view(user_id),
        rem_view(category_id), rem_view(mcc), rem_view(country),
        jnp.pad(W_merchant_id, ((0, 0), (0, 96))).reshape(-1, 32),
        jnp.pad(W_user_id, ((0, 0), (0, 96))).reshape(-1, 32),
        W_category_id.T, W_mcc.T, W_country.T,
    )
    return out5d.transpose(2, 4, 0, 1, 3).reshape(B, L, DTOT)


# speedup vs baseline: 1.0054x; 1.0054x over previous
import jax
import jax.numpy as jnp
from jax import lax
from jax.experimental import pallas as pl
from jax.experimental.pallas import tpu as pltpu
from jax.experimental.pallas import tpu_sc as plsc

B, L = 4096, 50
DTOT = 112
TJ = DTOT // 8

_info = plsc.get_sparse_core_info()
NC, NS = _info.num_cores, _info.num_subcores
NW = NC * NS


def _sc_body(mid_t, uid_t, cid_t, mcc_t, cty_t, Wm, Wu, Wct, Wmcct, Wctyt,
             out, iv0, iv1, iv2, iv3, iv4, tc_v, tmcc_v, tcty_v,
             bmA, buA, bmB, buB, tileA, tileB, gsemA, gsemB, wsemA, wsemB):
    wid = lax.axis_index("s") * NC + lax.axis_index("c")
    bcol = pl.ds(wid * 128, 128)

    pltpu.sync_copy(mid_t.at[:, bcol], iv0)
    pltpu.sync_copy(uid_t.at[:, bcol], iv1)
    pltpu.sync_copy(cid_t.at[:, bcol], iv2)
    pltpu.sync_copy(mcc_t.at[:, bcol], iv3)
    pltpu.sync_copy(cty_t.at[:, bcol], iv4)
    pltpu.sync_copy(Wct, tc_v)
    pltpu.sync_copy(Wmcct, tmcc_v)
    pltpu.sync_copy(Wctyt, tcty_v)

    lane = lax.iota(jnp.int32, 16)
    smalls = (tc_v, tmcc_v, tcty_v)
    siv = (iv2, iv3, iv4)

    def issue_gathers(l, bm, bu, gsem):
        pltpu.async_copy(Wm.at[iv0.at[l]], bm, gsem)
        pltpu.async_copy(Wu.at[iv1.at[l]], bu, gsem)

    def wait_gathers(l, bm, bu, gsem):
        pltpu.make_async_copy(Wm.at[iv0.at[l]], bm, gsem).wait()
        pltpu.make_async_copy(Wu.at[iv1.at[l]], bu, gsem).wait()

    rots = [(lane + j) & 15 for j in range(16)]

    def transpose_into(tile, bm, bu, l):
        def bchunk(bc, carry):
            row = bc * 16 + lane
            for tjoff, buf in ((0, bm), (4, bu)):
                for f0 in (0, 16):
                    for j in range(16):
                        f = f0 + rots[j]
                        v = plsc.load_gather(buf, [row, f])
                        plsc.store_scatter(
                            tile, [tjoff + (f >> 3), f & 7, row], v)
            return carry
        lax.fori_loop(0, 8, bchunk, 0)

        def chunk(k, carry):
            col16 = pl.ds(k * 16, 16)
            for t in range(3):
                idxv = siv[t][l, pl.ds(k * 16, 16)]
                for f in range(16):
                    v = plsc.load_gather(
                        smalls[t], [jnp.full((16,), f, jnp.int32), idxv])
                    tile[8 + 2 * t + f // 8, f % 8, col16] = v
            return carry
        lax.fori_loop(0, 8, chunk, 0)

    def issue_write(l, tile, wsem):
        pltpu.async_copy(tile, out.at[l, :, wid], wsem)

    def wait_write(l, tile, wsem):
        pltpu.make_async_copy(tile, out.at[l, :, wid], wsem).wait()

    issue_gathers(0, bmA, buA, gsemA)
    issue_gathers(1, bmB, buB, gsemB)

    def step(k, carry):
        l0, l1 = 2 * k, 2 * k + 1

        wait_gathers(l0, bmA, buA, gsemA)

        @pl.when(k > 0)
        def _():
            wait_write(l0, tileA, wsemA)
        transpose_into(tileA, bmA, buA, l0)
        issue_write(l0, tileA, wsemA)

        @pl.when(k < L // 2 - 1)
        def _():
            issue_gathers(l0 + 2, bmA, buA, gsemA)

        wait_gathers(l1, bmB, buB, gsemB)

        @pl.when(k > 0)
        def _():
            wait_write(l1, tileB, wsemB)
        transpose_into(tileB, bmB, buB, l1)
        issue_write(l1, tileB, wsemB)

        @pl.when(k < L // 2 - 1)
        def _():
            issue_gathers(l1 + 2, bmB, buB, gsemB)
        return carry

    lax.fori_loop(0, L // 2, step, 0)
    wait_write(L - 2, tileA, wsemA)
    wait_write(L - 1, tileB, wsemB)


@jax.jit
def kernel(merchant_id, user_id, category_id, mcc, country,
           W_merchant_id, W_user_id, W_category_id, W_mcc, W_country):
    mesh = plsc.VectorSubcoreMesh(core_axis_name="c", subcore_axis_name="s")
    run = pl.kernel(
        _sc_body,
        out_type=jax.ShapeDtypeStruct((L, TJ, NW, 8, 128), jnp.float32),
        mesh=mesh,
        scratch_types=(
            [pltpu.VMEM((L, 128), jnp.int32) for _ in range(5)]
            + [pltpu.VMEM((16, 1000), jnp.float32) for _ in range(3)]
            + [pltpu.VMEM((128, 32), jnp.float32) for _ in range(4)]
            + [pltpu.VMEM((TJ, 8, 128), jnp.float32) for _ in range(2)]
            + [pltpu.SemaphoreType.DMA] * 4
        ),
        compiler_params=pltpu.CompilerParams(use_tc_tiling_on_sc=False,
                                             needs_layout_passes=False),
    )
    out5d = run(
        merchant_id.T, user_id.T, category_id.T, mcc.T, country.T,
        W_merchant_id, W_user_id,
        W_category_id.T, W_mcc.T, W_country.T,
    )
    return out5d.transpose(2, 4, 0, 1, 3).reshape(B, L, DTOT)
